# Initial kernel scaffold; baseline (speedup 1.0000x reference)
import jax
import jax.numpy as jnp
from jax.experimental import pallas as pl


def kernel(x, text_state, Wqkv, bqkv, Wo, bo, rel_bias, ln1_g, ln1_b, Wr, Wt, ln2_g, ln2_b, W1, b1, W2, b2):
    raise NotImplementedError("write your pallas kernel here")


# speedup vs baseline: 21.4494x; 21.4494x over previous
import functools

import jax
import jax.numpy as jnp
from jax.experimental import pallas as pl
from jax.experimental.pallas import tpu as pltpu

S, D, H, E = 2048, 768, 12, 8
DH = D // H
F = 4 * D
BQ = 256
BT = 256
NT = S // BT
NF = 4
FB = F // NF
LANEPAD = 128

_INTERPRET = False


def _ln_qkv_body(x_ref, g_ref, b_ref, w_ref, bias_ref, out_ref):
    x = x_ref[...]
    m = jnp.mean(x, axis=-1, keepdims=True)
    v = jnp.mean((x - m) * (x - m), axis=-1, keepdims=True)
    h = (x - m) * jax.lax.rsqrt(v + 1e-5) * g_ref[...] + b_ref[...]
    out_ref[...] = (
        jnp.dot(h, w_ref[...], preferred_element_type=jnp.float32) + bias_ref[...]
    )


def _ln_qkv(x, g, b, w, bias):
    return pl.pallas_call(
        _ln_qkv_body,
        grid=(NT,),
        in_specs=[
            pl.BlockSpec((BT, D), lambda i: (i, 0)),
            pl.BlockSpec((1, D), lambda i: (0, 0)),
            pl.BlockSpec((1, D), lambda i: (0, 0)),
            pl.BlockSpec((D, 3 * D), lambda i: (0, 0)),
            pl.BlockSpec((1, 3 * D), lambda i: (0, 0)),
        ],
        out_specs=pl.BlockSpec((BT, 3 * D), lambda i: (i, 0)),
        out_shape=jax.ShapeDtypeStruct((S, 3 * D), jnp.float32),
        interpret=_INTERPRET,
    )(x, g, b, w, bias)


def _attn_body(q_ref, k_ref, v_ref, r_ref, o_ref):
    q = q_ref[0]
    k = k_ref[0]
    logits = jax.lax.dot_general(
        q, k, (((1,), (1,)), ((), ())), preferred_element_type=jnp.float32
    ) * (1.0 / 8.0)
    w = r_ref[0, 0, 0, :]
    m = jnp.broadcast_to(w[None, :], (BQ, BQ + S))
    row = jax.lax.broadcasted_iota(jnp.int32, (BQ, 1), 0)
    shift = 1
    while shift < BQ:
        rolled = pltpu.roll(m, shift, axis=1)
        m = jnp.where((row & shift) != 0, rolled, m)
        shift *= 2
    bias = m[:, BQ - 1 : BQ - 1 + S]
    logits = logits + bias
    mx = jnp.max(logits, axis=-1, keepdims=True)
    p = jnp.exp(logits - mx)
    a = p / jnp.sum(p, axis=-1, keepdims=True)
    o_ref[0] = jnp.dot(a, v_ref[0], preferred_element_type=jnp.float32)


def _attention(q, k, v, rrev):
    return pl.pallas_call(
        _attn_body,
        grid=(H, S // BQ),
        in_specs=[
            pl.BlockSpec((1, BQ, DH), lambda h, i: (h, i, 0)),
            pl.BlockSpec((1, S, DH), lambda h, i: (h, 0, 0)),
            pl.BlockSpec((1, S, DH), lambda h, i: (h, 0, 0)),
            pl.BlockSpec((1, 1, 1, BQ + S), lambda h, i: (h, i, 0, 0)),
        ],
        out_specs=pl.BlockSpec((1, BQ, DH), lambda h, i: (h, i, 0)),
        out_shape=jax.ShapeDtypeStruct((H, S, DH), jnp.float32),
        interpret=_INTERPRET,
    )(q, k, v, rrev)


def _proj_router_body(
    x_ref, o_ref, wo_ref, bo_ref, g2_ref, b2_ref, wr_ref, ts_ref, wt_ref,
    x2_ref, h2_ref, rl_ref,
):
    x2 = (
        x_ref[...]
        + jnp.dot(o_ref[...], wo_ref[...], preferred_element_type=jnp.float32)
        + bo_ref[...]
    )
    m = jnp.mean(x2, axis=-1, keepdims=True)
    v = jnp.mean((x2 - m) * (x2 - m), axis=-1, keepdims=True)
    h2 = (x2 - m) * jax.lax.rsqrt(v + 1e-5) * g2_ref[...] + b2_ref[...]
    tvec = jnp.dot(ts_ref[...], wt_ref[...], preferred_element_type=jnp.float32)
    rl = jnp.dot(h2, wr_ref[...], preferred_element_type=jnp.float32) + tvec
    x2_ref[...] = x2
    h2_ref[...] = h2
    rl_ref[...] = rl


def _proj_router(x, o, wo, bo, g2, b2, wr_pad, ts, wt_pad):
    return pl.pallas_call(
        _proj_router_body,
        grid=(NT,),
        in_specs=[
            pl.BlockSpec((BT, D), lambda i: (i, 0)),
            pl.BlockSpec((BT, D), lambda i: (i, 0)),
            pl.BlockSpec((D, D), lambda i: (0, 0)),
            pl.BlockSpec((1, D), lambda i: (0, 0)),
            pl.BlockSpec((1, D), lambda i: (0, 0)),
            pl.BlockSpec((1, D), lambda i: (0, 0)),
            pl.BlockSpec((D, LANEPAD), lambda i: (0, 0)),
            pl.BlockSpec((1, D), lambda i: (0, 0)),
            pl.BlockSpec((D, LANEPAD), lambda i: (0, 0)),
        ],
        out_specs=[
            pl.BlockSpec((BT, D), lambda i: (i, 0)),
            pl.BlockSpec((BT, D), lambda i: (i, 0)),
            pl.BlockSpec((BT, LANEPAD), lambda i: (i, 0)),
        ],
        out_shape=[
            jax.ShapeDtypeStruct((S, D), jnp.float32),
            jax.ShapeDtypeStruct((S, D), jnp.float32),
            jax.ShapeDtypeStruct((S, LANEPAD), jnp.float32),
        ],
        interpret=_INTERPRET,
    )(x, o, wo, bo, g2, b2, wr_pad, ts, wt_pad)


def _route_body(rl_ref, gates_ref, diag_ref):
    lane = jax.lax.broadcasted_iota(jnp.int32, (S, LANEPAD), 1)
    valid = lane < E
    z = jnp.where(valid, rl_ref[...], -1e30)
    z = z - jnp.max(z, axis=-1, keepdims=True)
    ez = jnp.where(valid, jnp.exp(z), 0.0)
    p = ez / jnp.sum(ez, axis=-1, keepdims=True)
    m1 = jnp.max(p, axis=-1, keepdims=True)
    i1 = jnp.min(jnp.where((p == m1) & valid, lane, LANEPAD), axis=-1, keepdims=True)
    p2 = jnp.where(lane == i1, -1.0, p)
    m2 = jnp.max(p2, axis=-1, keepdims=True)
    i2 = jnp.min(jnp.where((p2 == m2) & valid, lane, LANEPAD), axis=-1, keepdims=True)
    tot = m1 + m2
    gates = jnp.where(lane == i1, m1 / tot, 0.0) + jnp.where(lane == i2, m2 / tot, 0.0)
    gates_ref[...] = gates
    diag_ref[...] = jnp.mean(gates, axis=0, keepdims=True)


def _route(rlog):
    return pl.pallas_call(
        _route_body,
        out_shape=[
            jax.ShapeDtypeStruct((S, LANEPAD), jnp.float32),
            jax.ShapeDtypeStruct((1, LANEPAD), jnp.float32),
        ],
        interpret=_INTERPRET,
    )(rlog)


def _moe_body(h2_ref, x2_ref, gt_ref, w1_ref, b1_ref, w2_ref, b2_ref, out_ref):
    e = pl.program_id(0)
    f = pl.program_id(1)

    @pl.when((e == 0) & (f == 0))
    def _init():
        out_ref[...] = x2_ref[...]

    w1 = w1_ref[0]
    w2 = w2_ref[0]
    b1 = b1_ref[0]
    b2s = jnp.where(f == 0, 1.0, 0.0) * b2_ref[0]
    g_row = gt_ref[0, 0]
    for tb in range(NT):
        h2 = h2_ref[pl.ds(tb * BT, BT), :]
        h1 = jnp.dot(h2, w1, preferred_element_type=jnp.float32) + b1
        h1 = jax.nn.gelu(h1)
        eo = jnp.dot(h1, w2, preferred_element_type=jnp.float32) + b2s
        g = g_row[tb * BT : (tb + 1) * BT][:, None]
        out_ref[pl.ds(tb * BT, BT), :] += g * eo


def _moe(h2, x2, gates_t, w1, b1, w2, b2):
    return pl.pallas_call(
        _moe_body,
        grid=(E, NF),
        in_specs=[
            pl.BlockSpec((S, D), lambda e, f: (0, 0)),
            pl.BlockSpec((S, D), lambda e, f: (0, 0)),
            pl.BlockSpec((1, 1, S), lambda e, f: (e, 0, 0)),
            pl.BlockSpec((1, D, FB), lambda e, f: (e, 0, f)),
            pl.BlockSpec((1, 1, FB), lambda e, f: (e, 0, f)),
            pl.BlockSpec((1, FB, D), lambda e, f: (e, f, 0)),
            pl.BlockSpec((1, 1, D), lambda e, f: (e, 0, 0)),
        ],
        out_specs=pl.BlockSpec((S, D), lambda e, f: (0, 0)),
        out_shape=jax.ShapeDtypeStruct((S, D), jnp.float32),
        interpret=_INTERPRET,
    )(h2, x2, gates_t, w1, b1, w2, b2)


@jax.jit
def _run(x, text_state, Wqkv, bqkv, Wo, bo, rel_bias, ln1_g, ln1_b, Wr, Wt,
         ln2_g, ln2_b, W1, b1, W2, b2):
    x2d = x[0]
    qkv = _ln_qkv(
        x2d, ln1_g.reshape(1, D), ln1_b.reshape(1, D), Wqkv, bqkv.reshape(1, 3 * D)
    )
    q = qkv[:, :D].reshape(S, H, DH).transpose(1, 0, 2)
    k = qkv[:, D : 2 * D].reshape(S, H, DH).transpose(1, 0, 2)
    v = qkv[:, 2 * D :].reshape(S, H, DH).transpose(1, 0, 2)
    rrev = jnp.pad(rel_bias[:, ::-1], ((0, 0), (0, 1)))
    rwin = jnp.stack(
        [rrev[:, (NT - 1 - qi) * BQ : (NT - 1 - qi) * BQ + BQ + S]
         for qi in range(S // BQ)],
        axis=1,
    )[:, :, None, :]
    o = _attention(q, k, v, rwin)
    o2 = o.transpose(1, 0, 2).reshape(S, D)
    wr_pad = jnp.pad(Wr, ((0, 0), (0, LANEPAD - E)))
    wt_pad = jnp.pad(Wt, ((0, 0), (0, LANEPAD - E)))
    x2, h2, rlog = _proj_router(
        x2d, o2, Wo, bo.reshape(1, D), ln2_g.reshape(1, D), ln2_b.reshape(1, D),
        wr_pad, text_state, wt_pad,
    )
    gates, diag = _route(rlog)
    gates_t = gates[:, :E].T.reshape(E, 1, S)
    y = _moe(h2, x2, gates_t, W1, b1.reshape(E, 1, F), W2, b2.reshape(E, 1, D))
    return y[None], diag[0, :E]


def kernel(x, text_state, Wqkv, bqkv, Wo, bo, rel_bias, ln1_g, ln1_b, Wr, Wt,
           ln2_g, ln2_b, W1, b1, W2, b2):
    return _run(x, text_state, Wqkv, bqkv, Wo, bo, rel_bias, ln1_g, ln1_b,
                Wr, Wt, ln2_g, ln2_b, W1, b1, W2, b2)
